# Initial kernel scaffold; baseline (speedup 1.0000x reference)
import jax
import jax.numpy as jnp
from jax.experimental import pallas as pl


def kernel(up_row, up_col, reg_row, reg_poi, cat_row, cat_poi, src_poi, src_he, tar_he, tar_poi, user_idx, poi_emb, user_emb, region_emb, category_emb, w_gate_col, b_gate_col, gate_col_W, gate_col_b, w_gate_trans, b_gate_trans, gate_trans_W, gate_trans_b, w_gate_reg, b_gate_reg, gate_reg_W, gate_reg_b, w_gate_cat, b_gate_cat, gate_cat_W, gate_cat_b, col_Wp, col_We, col_Wf, reg_Wp, reg_We, reg_Wf, cat_Wp, cat_We, cat_Wf):
    raise NotImplementedError("write your pallas kernel here")


# speedup vs baseline: 2.6969x; 2.6969x over previous
import functools

import jax
import jax.numpy as jnp
from jax import lax
from jax.experimental import pallas as pl
from jax.experimental.pallas import tpu as pltpu
from jax.experimental.pallas import tpu_sc as plsc

_NC = 2
_NS = 16
_W = 128
_EDGE_ALIGN = _NS * _W
_ROW_ALIGN = 256
_NT = 20000

_PREC = lax.Precision.HIGHEST


def _rup(n, m):
    return (n + m - 1) // m * m


@functools.lru_cache(maxsize=None)
def _spmm_kernel(E_pad, R_pad, M):
    chunks = E_pad // _EDGE_ALIGN
    rows_pt = R_pad // _NS
    mesh = plsc.VectorSubcoreMesh(core_axis_name="c", subcore_axis_name="s",
                                  num_cores=_NC, num_subcores=_NS)

    @functools.partial(
        pl.kernel,
        out_type=jax.ShapeDtypeStruct((R_pad, 128), jnp.float32),
        mesh=mesh,
        compiler_params=pltpu.CompilerParams(use_tc_tiling_on_sc=False),
        scratch_types=[
            pltpu.VMEM((_W,), jnp.int32),
            pltpu.VMEM((_W,), jnp.int32),
            pltpu.VMEM((_W, 64), jnp.float32),
            pltpu.VMEM((16, 64), jnp.float32),
            pltpu.VMEM_SHARED((R_pad, 64), jnp.float32),
            pltpu.SemaphoreType.DMA,
        ],
    )
    def k(x_hbm, g_hbm, s_hbm, out_hbm, gix, six, rows, zb, acc, sem):
        c = lax.axis_index("c")
        s = lax.axis_index("s")

        @pl.loop(0, 16)
        def _zrow(i):
            @pl.loop(0, 4)
            def _zcol(j):
                zb[i, pl.ds(j * 16, 16)] = jnp.zeros((16,), jnp.float32)

        r0 = s * rows_pt

        @pl.loop(0, rows_pt, step=16)
        def _zacc(r):
            pltpu.sync_copy(zb, acc.at[pl.ds(r0 + r, 16)])

        plsc.subcore_barrier()

        e0 = s * (chunks * _W)

        @pl.loop(0, chunks)
        def _edge(t):
            base = e0 + t * _W
            pltpu.sync_copy(g_hbm.at[c, pl.ds(base, _W)], gix)
            pltpu.sync_copy(s_hbm.at[pl.ds(base, _W)], six)
            pltpu.async_copy(x_hbm.at[gix], rows, sem).wait()
            pltpu.sync_copy(rows, acc.at[six], add=True)

        plsc.subcore_barrier()

        for cc in range(_NC):
            @pl.when(c == cc)
            def _out():
                pltpu.sync_copy(
                    acc.at[pl.ds(r0, rows_pt)],
                    out_hbm.at[pl.ds(r0, rows_pt), pl.ds(cc * 64, 64)])

    return k


def _sc_segment(jobs):
    xs, offs = [], []
    m = 0
    for x, _, _, _ in jobs:
        xs.append(x.reshape(-1, 64))
        offs.append(m)
        m += 2 * x.shape[0]
    xp = jnp.concatenate(xs, axis=0) if len(xs) > 1 else xs[0]

    gs, ss, bounds = [], [], []
    r = 0
    for (x, gcol, srow, n_out), bx in zip(jobs, offs):
        gs.append(2 * gcol + bx)
        ss.append(srow + r)
        bounds.append((r, n_out))
        r += _rup(n_out + 1, _ROW_ALIGN)
    g = jnp.concatenate(gs) if len(gs) > 1 else gs[0]
    sx = jnp.concatenate(ss) if len(ss) > 1 else ss[0]

    E = g.shape[0]
    E_pad = _rup(E, _EDGE_ALIGN)
    if E_pad != E:
        dump = bounds[-1][0] + bounds[-1][1]
        g = jnp.concatenate([g, jnp.zeros((E_pad - E,), jnp.int32)])
        sx = jnp.concatenate([sx, jnp.full((E_pad - E,), dump, jnp.int32)])
    g2 = jnp.stack([g, g + 1])

    out = _spmm_kernel(E_pad, r, xp.shape[0])(xp, g2, sx)
    return [out[r0:r0 + n] for r0, n in bounds]


def _blk(n):
    if n <= 512:
        return n
    for b in (1000, 512, 400, 200, 8):
        if n % b == 0:
            return b
    return n


def _tc_gate(base, Wg, bg, Wp3):
    n = base.shape[0]
    rb = _blk(n)

    def body(b_ref, wg_ref, bg_ref, wp_ref, gated_ref, pw_ref):
        b = b_ref[...]
        gs = []
        for i in range(4):
            z = jnp.dot(b, wg_ref[i], preferred_element_type=jnp.float32,
                        precision=_PREC) + bg_ref[i]
            g = b * jax.nn.sigmoid(z)
            gs.append(g)
            gated_ref[i] = g
        for j, i in enumerate((0, 2, 3)):
            pw_ref[j] = jnp.dot(gs[i], wp_ref[j],
                                preferred_element_type=jnp.float32,
                                precision=_PREC)

    return pl.pallas_call(
        body,
        grid=(n // rb,),
        in_specs=[
            pl.BlockSpec((rb, 128), lambda i: (i, 0)),
            pl.BlockSpec((4, 128, 128), lambda i: (0, 0, 0)),
            pl.BlockSpec((4, 1, 128), lambda i: (0, 0, 0)),
            pl.BlockSpec((3, 128, 128), lambda i: (0, 0, 0)),
        ],
        out_specs=[
            pl.BlockSpec((4, rb, 128), lambda i: (0, i, 0)),
            pl.BlockSpec((3, rb, 128), lambda i: (0, i, 0)),
        ],
        out_shape=[
            jax.ShapeDtypeStruct((4, n, 128), jnp.float32),
            jax.ShapeDtypeStruct((3, n, 128), jnp.float32),
        ],
    )(base, Wg, bg, Wp3)


def _tc_fuse(pm, e, We, Wf_top, Wf_bot):
    n = e.shape[0]
    rb = _blk(n)

    def body(pm_ref, e_ref, we_ref, wt_ref, wb_ref, f_ref, en_ref):
        ev = e_ref[...]
        em = jnp.dot(ev, we_ref[...], preferred_element_type=jnp.float32,
                     precision=_PREC)
        f = (jnp.dot(pm_ref[...], wt_ref[...],
                     preferred_element_type=jnp.float32, precision=_PREC)
             + jnp.dot(em, wb_ref[...], preferred_element_type=jnp.float32,
                       precision=_PREC))
        f_ref[...] = f
        en_ref[...] = f + ev

    return pl.pallas_call(
        body,
        grid=(n // rb,),
        in_specs=[
            pl.BlockSpec((rb, 128), lambda i: (i, 0)),
            pl.BlockSpec((rb, 128), lambda i: (i, 0)),
            pl.BlockSpec((128, 128), lambda i: (0, 0)),
            pl.BlockSpec((128, 128), lambda i: (0, 0)),
            pl.BlockSpec((128, 128), lambda i: (0, 0)),
        ],
        out_specs=[
            pl.BlockSpec((rb, 128), lambda i: (i, 0)),
            pl.BlockSpec((rb, 128), lambda i: (i, 0)),
        ],
        out_shape=[
            jax.ShapeDtypeStruct((n, 128), jnp.float32),
            jax.ShapeDtypeStruct((n, 128), jnp.float32),
        ],
    )(pm, e, We, Wf_top, Wf_bot)


def _tc_resid_w(prop, p_prev, Wp):
    n = prop.shape[0]
    rb = _blk(n)

    def body(pr_ref, pp_ref, w_ref, p_ref, pw_ref):
        p = pr_ref[...] + pp_ref[...]
        p_ref[...] = p
        pw_ref[...] = jnp.dot(p, w_ref[...],
                              preferred_element_type=jnp.float32,
                              precision=_PREC)

    return pl.pallas_call(
        body,
        grid=(n // rb,),
        in_specs=[
            pl.BlockSpec((rb, 128), lambda i: (i, 0)),
            pl.BlockSpec((rb, 128), lambda i: (i, 0)),
            pl.BlockSpec((128, 128), lambda i: (0, 0)),
        ],
        out_specs=[
            pl.BlockSpec((rb, 128), lambda i: (i, 0)),
            pl.BlockSpec((rb, 128), lambda i: (i, 0)),
        ],
        out_shape=[
            jax.ShapeDtypeStruct((n, 128), jnp.float32),
            jax.ShapeDtypeStruct((n, 128), jnp.float32),
        ],
    )(prop, p_prev, Wp)


def _tc_add(a, b):
    n = a.shape[0]
    rb = _blk(n)

    def body(a_ref, b_ref, o_ref):
        o_ref[...] = a_ref[...] + b_ref[...]

    return pl.pallas_call(
        body,
        grid=(n // rb,),
        in_specs=[pl.BlockSpec((rb, 128), lambda i: (i, 0)),
                  pl.BlockSpec((rb, 128), lambda i: (i, 0))],
        out_specs=pl.BlockSpec((rb, 128), lambda i: (i, 0)),
        out_shape=jax.ShapeDtypeStruct((n, 128), jnp.float32),
    )(a, b)


def _tc_user_mean(e0, f1, f2):
    n = e0.shape[0]
    rb = _blk(n)

    def body(e_ref, f1_ref, f2_ref, o_ref):
        o_ref[...] = e_ref[...] + (2.0 * f1_ref[...] + f2_ref[...]) * (1.0 / 3.0)

    return pl.pallas_call(
        body,
        grid=(n // rb,),
        in_specs=[pl.BlockSpec((rb, 128), lambda i: (i, 0))] * 3,
        out_specs=pl.BlockSpec((rb, 128), lambda i: (i, 0)),
        out_shape=jax.ShapeDtypeStruct((n, 128), jnp.float32),
    )(e0, f1, f2)


def _normalize_blk(x):
    nrm = jnp.sqrt(jnp.sum(x * x, axis=-1, keepdims=True))
    return x / jnp.maximum(nrm, 1e-12)


def _tc_poi_final(P0, P1, PR):
    n = P0.shape[1]
    rb = _blk(n)

    def body(p0_ref, p1_ref, pr_ref, poi_ref, fin_ref):
        acc = jnp.zeros((rb, 128), jnp.float32)
        for b in range(4):
            p0 = p0_ref[b]
            p1 = p1_ref[b]
            p2 = pr_ref[b] + p1
            po = (p0 + p1 + p2) * (1.0 / 3.0)
            poi_ref[b] = po
            acc = acc + _normalize_blk(po)
        fin_ref[...] = acc

    return pl.pallas_call(
        body,
        grid=(n // rb,),
        in_specs=[pl.BlockSpec((4, rb, 128), lambda i: (0, i, 0))] * 3,
        out_specs=[
            pl.BlockSpec((4, rb, 128), lambda i: (0, i, 0)),
            pl.BlockSpec((rb, 128), lambda i: (i, 0)),
        ],
        out_shape=[
            jax.ShapeDtypeStruct((4, n, 128), jnp.float32),
            jax.ShapeDtypeStruct((n, 128), jnp.float32),
        ],
    )(P0, P1, PR)


def _tc_user_final(U, GW, GB):
    n = U.shape[1]
    rb = _blk(n)

    def body(u_ref, gw_ref, gb_ref, o_ref):
        acc = jnp.zeros((rb, 128), jnp.float32)
        for b in range(4):
            nb = _normalize_blk(u_ref[b])
            s = jnp.sum(nb * gw_ref[b], axis=-1, keepdims=True)
            coef = jax.nn.sigmoid(s + gb_ref[b][0, 0])
            acc = acc + coef * nb
        o_ref[...] = acc

    return pl.pallas_call(
        body,
        grid=(n // rb,),
        in_specs=[
            pl.BlockSpec((4, rb, 128), lambda i: (0, i, 0)),
            pl.BlockSpec((4, 1, 128), lambda i: (0, 0, 0)),
            pl.BlockSpec((4, 1, 128), lambda i: (0, 0, 0)),
        ],
        out_specs=pl.BlockSpec((rb, 128), lambda i: (i, 0)),
        out_shape=jax.ShapeDtypeStruct((n, 128), jnp.float32),
    )(U, GW, GB)


def kernel(up_row, up_col, reg_row, reg_poi, cat_row, cat_poi,
           src_poi, src_he, tar_he, tar_poi, user_idx,
           poi_emb, user_emb, region_emb, category_emb,
           w_gate_col, b_gate_col, gate_col_W, gate_col_b,
           w_gate_trans, b_gate_trans, gate_trans_W, gate_trans_b,
           w_gate_reg, b_gate_reg, gate_reg_W, gate_reg_b,
           w_gate_cat, b_gate_cat, gate_cat_W, gate_cat_b,
           col_Wp, col_We, col_Wf,
           reg_Wp, reg_We, reg_Wf,
           cat_Wp, cat_We, cat_Wf):
    n_poi = poi_emb.shape[0] - 1
    n_user = user_emb.shape[0]
    n_reg = region_emb.shape[0]
    n_cat = category_emb.shape[0]
    batch = user_idx.shape[0]
    base = poi_emb[:n_poi]

    Wg = jnp.stack([w_gate_col, w_gate_trans, w_gate_reg, w_gate_cat])
    bg = jnp.stack([b_gate_col, b_gate_trans, b_gate_reg, b_gate_cat])
    Wp3 = jnp.stack([col_Wp, reg_Wp, cat_Wp])
    gated, pw = _tc_gate(base, Wg, bg, Wp3)
    col_p0, trans_p0, reg_p0, cat_p0 = (gated[0], gated[1], gated[2], gated[3])
    colPW, regPW, catPW = pw[0], pw[1], pw[2]

    col_Wt, col_Wb = col_Wf[:128], col_Wf[128:]
    reg_Wt, reg_Wb = reg_Wf[:128], reg_Wf[128:]
    cat_Wt, cat_Wb = cat_Wf[:128], cat_Wf[128:]

    col_pm1, reg_pm1, cat_pm1, mt1 = _sc_segment([
        (colPW, up_col, up_row, n_user),
        (regPW, reg_poi, reg_row, n_reg),
        (catPW, cat_poi, cat_row, n_cat),
        (trans_p0, tar_poi, tar_he, _NT),
    ])
    col_f1, col_e1 = _tc_fuse(col_pm1, user_emb, col_We, col_Wt, col_Wb)
    reg_f1, reg_e1 = _tc_fuse(reg_pm1, region_emb, reg_We, reg_Wt, reg_Wb)
    cat_f1, cat_e1 = _tc_fuse(cat_pm1, category_emb, cat_We, cat_Wt, cat_Wb)

    col_pr1, reg_pr1 = _sc_segment([
        (col_f1, up_row, up_col, n_poi),
        (reg_f1, reg_row, reg_poi, n_poi),
    ])
    cat_pr1, ms1 = _sc_segment([
        (cat_f1, cat_row, cat_poi, n_poi),
        (mt1, src_he, src_poi, n_poi),
    ])

    col_p1, colPW2 = _tc_resid_w(col_pr1, col_p0, col_Wp)
    reg_p1, regPW2 = _tc_resid_w(reg_pr1, reg_p0, reg_Wp)
    cat_p1, catPW2 = _tc_resid_w(cat_pr1, cat_p0, cat_Wp)
    trans_p1 = _tc_add(ms1, trans_p0)

    col_pm2, reg_pm2, cat_pm2, mt2 = _sc_segment([
        (colPW2, up_col, up_row, n_user),
        (regPW2, reg_poi, reg_row, n_reg),
        (catPW2, cat_poi, cat_row, n_cat),
        (trans_p1, tar_poi, tar_he, _NT),
    ])
    (ms2,) = _sc_segment([(mt2, src_he, src_poi, n_poi)])

    col_f2, col_e2 = _tc_fuse(col_pm2, col_e1, col_We, col_Wt, col_Wb)
    reg_f2, reg_e2 = _tc_fuse(reg_pm2, reg_e1, reg_We, reg_Wt, reg_Wb)
    cat_f2, cat_e2 = _tc_fuse(cat_pm2, cat_e1, cat_We, cat_Wt, cat_Wb)

    col_pr2, reg_pr2 = _sc_segment([
        (col_f2, up_row, up_col, n_poi),
        (reg_f2, reg_row, reg_poi, n_poi),
    ])
    cat_pr2, ms2 = _sc_segment([
        (cat_f2, cat_row, cat_poi, n_poi),
        (mt2, src_he, src_poi, n_poi),
    ])

    P0 = jnp.stack([col_p0, reg_p0, cat_p0, trans_p0])
    P1 = jnp.stack([col_p1, reg_p1, cat_p1, trans_p1])
    PR = jnp.stack([col_pr2, reg_pr2, cat_pr2, ms2])
    poi_out, final_poi = _tc_poi_final(P0, P1, PR)

    col_user = _tc_user_mean(user_emb, col_f1, col_f2)
    reg_user, cat_user, trans_user = _sc_segment([
        (poi_out[1], up_col, up_row, n_user),
        (poi_out[2], up_col, up_row, n_user),
        (poi_out[3], up_col, up_row, n_user),
    ])

    ar = jnp.arange(batch, dtype=jnp.int32)
    cu, ru, au, tu = _sc_segment([
        (col_user, user_idx, ar, batch),
        (reg_user, user_idx, ar, batch),
        (cat_user, user_idx, ar, batch),
        (trans_user, user_idx, ar, batch),
    ])

    U = jnp.stack([cu, ru, au, tu])
    GW = jnp.stack([gate_col_W.reshape(1, 128), gate_reg_W.reshape(1, 128),
                    gate_cat_W.reshape(1, 128), gate_trans_W.reshape(1, 128)])
    GB = jnp.stack([jnp.broadcast_to(b.reshape(1, 1), (1, 128))
                    for b in (gate_col_b, gate_reg_b, gate_cat_b, gate_trans_b)])
    final_user = _tc_user_final(U, GW, GB)
    return (final_user, final_poi)
